# Initial kernel scaffold; baseline (speedup 1.0000x reference)
import functools

import jax
import jax.numpy as jnp
from jax import lax
from jax.experimental import pallas as pl
from jax.experimental.pallas import tpu as pltpu
from jax.experimental.pallas import tpu_sc as plsc

_N = 10000
_NPAD = 10016
_E = 320000
_D1 = 128
_DH = _D1 // 2
_D2 = 16
_C = 128
_NCORES = 2
_NSUB = 16
_ROWS_PER_TILE = _NPAD // _NSUB

_NCH1 = 158
_ALLOC1 = _NCH1 + 2
_NCH2 = 80
_ALLOC2 = _NCH2 + 2

_MESH = plsc.VectorSubcoreMesh(
    core_axis_name="c", subcore_axis_name="s",
    num_cores=_NCORES, num_subcores=_NSUB)


def _pipeline(table, srcv, dstv, acc, buf0, buf1, sem0, sem1, n_pairs):
  pltpu.async_copy(table.at[srcv.at[0]], buf0, sem0)

  def body(i, carry):
    j0 = 2 * i
    d1 = pltpu.async_copy(table.at[srcv.at[j0 + 1]], buf1, sem1)
    pltpu.make_async_copy(table.at[srcv.at[j0]], buf0, sem0).wait()
    pltpu.sync_copy(buf0, acc.at[dstv.at[j0]], add=True)
    pltpu.async_copy(table.at[srcv.at[j0 + 2]], buf0, sem0)
    d1.wait()
    pltpu.sync_copy(buf1, acc.at[dstv.at[j0 + 1]], add=True)
    return carry

  lax.fori_loop(0, n_pairs, body, 0)
  pltpu.make_async_copy(table.at[srcv.at[2 * n_pairs]], buf0, sem0).wait()


def _agg1_body(x_hbm, src_hbm, dst_hbm, zeros_hbm, out_hbm,
               xsh, acc, srcv, dstv, buf0, buf1, sem0, sem1):
  c = lax.axis_index("c")
  s = lax.axis_index("s")
  r0 = s * _ROWS_PER_TILE
  col0 = c * _DH
  pltpu.sync_copy(x_hbm.at[pl.ds(r0, _ROWS_PER_TILE), pl.ds(col0, _DH)],
                  xsh.at[pl.ds(r0, _ROWS_PER_TILE)])
  pltpu.sync_copy(zeros_hbm.at[pl.ds(r0, _ROWS_PER_TILE)],
                  acc.at[pl.ds(r0, _ROWS_PER_TILE)])
  pltpu.sync_copy(src_hbm.at[s], srcv)
  pltpu.sync_copy(dst_hbm.at[s], dstv)
  plsc.subcore_barrier()
  _pipeline(xsh, srcv, dstv, acc, buf0, buf1, sem0, sem1, _NCH1 // 2)
  plsc.subcore_barrier()
  pltpu.sync_copy(acc.at[pl.ds(r0, _ROWS_PER_TILE)],
                  out_hbm.at[pl.ds(r0, _ROWS_PER_TILE), pl.ds(col0, _DH)])


_agg1 = functools.partial(
    pl.kernel, _agg1_body,
    out_type=jax.ShapeDtypeStruct((_NPAD, _D1), jnp.float32),
    mesh=_MESH,
    scratch_types=[
        pltpu.VMEM_SHARED((_NPAD, _DH), jnp.float32),
        pltpu.VMEM_SHARED((_NPAD, _DH), jnp.float32),
        pltpu.VMEM((_ALLOC1, _C), jnp.int32),
        pltpu.VMEM((_ALLOC1, _C), jnp.int32),
        pltpu.VMEM((_C, _DH), jnp.float32),
        pltpu.VMEM((_C, _DH), jnp.float32),
        pltpu.SemaphoreType.DMA,
        pltpu.SemaphoreType.DMA,
    ])()


def _agg2_body(h_hbm, src_hbm, dst_hbm, zeros_hbm, out_hbm,
               hsh, acc, srcv, dstv, buf0, buf1, sem0, sem1):
  c = lax.axis_index("c")
  s = lax.axis_index("s")
  r0 = s * _ROWS_PER_TILE
  pltpu.sync_copy(h_hbm.at[pl.ds(r0, _ROWS_PER_TILE)],
                  hsh.at[pl.ds(r0, _ROWS_PER_TILE)])
  pltpu.sync_copy(zeros_hbm.at[pl.ds(r0, _ROWS_PER_TILE)],
                  acc.at[pl.ds(r0, _ROWS_PER_TILE)])
  pltpu.sync_copy(src_hbm.at[c, s], srcv)
  pltpu.sync_copy(dst_hbm.at[c, s], dstv)
  plsc.subcore_barrier()
  _pipeline(hsh, srcv, dstv, acc, buf0, buf1, sem0, sem1, _NCH2 // 2)
  plsc.subcore_barrier()
  pltpu.sync_copy(acc.at[pl.ds(r0, _ROWS_PER_TILE)],
                  out_hbm.at[c, pl.ds(r0, _ROWS_PER_TILE)])


_agg2 = functools.partial(
    pl.kernel, _agg2_body,
    out_type=jax.ShapeDtypeStruct((_NCORES, _NPAD, _D2), jnp.float32),
    mesh=_MESH,
    scratch_types=[
        pltpu.VMEM_SHARED((_NPAD, _D2), jnp.float32),
        pltpu.VMEM_SHARED((_NPAD, _D2), jnp.float32),
        pltpu.VMEM((_ALLOC2, _C), jnp.int32),
        pltpu.VMEM((_ALLOC2, _C), jnp.int32),
        pltpu.VMEM((_C, _D2), jnp.float32),
        pltpu.VMEM((_C, _D2), jnp.float32),
        pltpu.SemaphoreType.DMA,
        pltpu.SemaphoreType.DMA,
    ])()


def _mm_body(a_ref, w1_ref, w2_ref, o_ref):
  wc = jnp.dot(w1_ref[...], w2_ref[...],
               preferred_element_type=jnp.float32,
               precision=lax.Precision.HIGHEST)
  o_ref[...] = jnp.dot(a_ref[...], wc,
                       preferred_element_type=jnp.float32,
                       precision=lax.Precision.HIGHEST)


_mm = pl.pallas_call(
    _mm_body, out_shape=jax.ShapeDtypeStruct((_NPAD, _D2), jnp.float32))


def _add_body(a_ref, b_ref, o_ref):
  o_ref[...] = a_ref[...] + b_ref[...]


_add = pl.pallas_call(
    _add_body, out_shape=jax.ShapeDtypeStruct((_NPAD, _D2), jnp.float32))


def _edge_layout(src, dst, lead_shape, nch_proc, alloc):
  n_tiles = 1
  for d in lead_shape:
    n_tiles *= d
  cap = n_tiles * nch_proc * _C
  npad = cap - src.shape[0]
  src_p = jnp.concatenate(
      [src, jnp.full((npad,), _N, jnp.int32)]).reshape(
          *lead_shape, nch_proc, _C)
  dst_p = jnp.concatenate(
      [dst, jnp.arange(npad, dtype=jnp.int32) % _NPAD]).reshape(
          *lead_shape, nch_proc, _C)
  extra = alloc - nch_proc
  src_p = jnp.concatenate(
      [src_p, jnp.full((*lead_shape, extra, _C), _N, jnp.int32)], axis=-2)
  dst_p = jnp.concatenate(
      [dst_p, jnp.zeros((*lead_shape, extra, _C), jnp.int32)], axis=-2)
  return src_p, dst_p


@jax.jit
def kernel(x, edge_index, W1, W2):
  src = edge_index[0].astype(jnp.int32)
  dst = edge_index[1].astype(jnp.int32)
  x_pad = jnp.zeros((_NPAD, _D1), jnp.float32).at[:_N].set(x)

  src1, dst1 = _edge_layout(src, dst, (_NSUB,), _NCH1, _ALLOC1)
  zeros1 = jnp.zeros((_NPAD, _DH), jnp.float32)
  agg1 = _agg1(x_pad, src1, dst1, zeros1)

  h2 = _mm(agg1, W1, W2)

  src2, dst2 = _edge_layout(src, dst, (_NCORES, _NSUB), _NCH2, _ALLOC2)
  zeros2 = jnp.zeros((_NPAD, _D2), jnp.float32)
  parts = _agg2(h2, src2, dst2, zeros2)

  out = _add(parts[0], parts[1])
  return out[:_N]


# speedup vs baseline: 10.9361x; 10.9361x over previous
import functools

import jax
import jax.numpy as jnp
from jax import lax
from jax.experimental import pallas as pl
from jax.experimental.pallas import tpu as pltpu
from jax.experimental.pallas import tpu_sc as plsc

_N = 10000
_NPAD = 10112
_E = 320000
_D1 = 128
_DH = _D1 // 2
_D2 = 16
_NCORES = 2
_NSUB = 16
_ROWS_PER_TILE = _NPAD // _NSUB

_C1 = 64
_NCH1 = 314
_ALLOC1 = _NCH1 + 2
_C2 = 128
_NCH2 = 80
_ALLOC2 = _NCH2 + 2

def _pipeline(table, srcv, dstv, acc, buf0, buf1, sem0, sem1, n_pairs):
  pltpu.async_copy(table.at[srcv.at[0]], buf0, sem0)

  def body(i, carry):
    j0 = 2 * i
    d1 = pltpu.async_copy(table.at[srcv.at[j0 + 1]], buf1, sem1)
    pltpu.make_async_copy(table.at[srcv.at[j0]], buf0, sem0).wait()
    pltpu.sync_copy(buf0, acc.at[dstv.at[j0]], add=True)
    pltpu.async_copy(table.at[srcv.at[j0 + 2]], buf0, sem0)
    d1.wait()
    pltpu.sync_copy(buf1, acc.at[dstv.at[j0 + 1]], add=True)
    return carry

  lax.fori_loop(0, n_pairs, body, 0)
  pltpu.make_async_copy(table.at[srcv.at[2 * n_pairs]], buf0, sem0).wait()


def _agg1_body(x_hbm, src_hbm, dst_hbm, zeros_hbm, out_hbm,
               xsh, acc, srcv, dstv, buf0, buf1, sem0, sem1):
  c = lax.axis_index("c")
  s = lax.axis_index("s")
  r0 = s * _ROWS_PER_TILE
  pltpu.sync_copy(x_hbm.at[c, pl.ds(r0, _ROWS_PER_TILE)],
                  xsh.at[pl.ds(r0, _ROWS_PER_TILE)])
  pltpu.sync_copy(zeros_hbm.at[pl.ds(r0, _ROWS_PER_TILE)],
                  acc.at[pl.ds(r0, _ROWS_PER_TILE)])
  pltpu.sync_copy(src_hbm.at[s], srcv)
  pltpu.sync_copy(dst_hbm.at[s], dstv)
  plsc.subcore_barrier()
  _pipeline(xsh, srcv, dstv, acc, buf0, buf1, sem0, sem1, _NCH1 // 2)
  plsc.subcore_barrier()
  pltpu.sync_copy(acc.at[pl.ds(r0, _ROWS_PER_TILE)],
                  out_hbm.at[c, pl.ds(r0, _ROWS_PER_TILE)])


@functools.cache
def _build_aggs():
  mesh = plsc.VectorSubcoreMesh(
      core_axis_name="c", subcore_axis_name="s",
      num_cores=_NCORES, num_subcores=_NSUB)
  params = pltpu.CompilerParams(use_tc_tiling_on_sc=False)
  agg1 = pl.kernel(
      _agg1_body,
      out_type=jax.ShapeDtypeStruct((_NCORES, _NPAD, _DH), jnp.float32),
      mesh=mesh,
      scratch_types=[
          pltpu.VMEM_SHARED((_NPAD, _DH), jnp.float32),
          pltpu.VMEM_SHARED((_NPAD, _DH), jnp.float32),
          pltpu.VMEM((_ALLOC1, _C1), jnp.int32),
          pltpu.VMEM((_ALLOC1, _C1), jnp.int32),
          pltpu.VMEM((_C1, _DH), jnp.float32),
          pltpu.VMEM((_C1, _DH), jnp.float32),
          pltpu.SemaphoreType.DMA,
          pltpu.SemaphoreType.DMA,
      ],
      compiler_params=params)
  agg2 = pl.kernel(
      _agg2_body,
      out_type=jax.ShapeDtypeStruct((_NCORES, _NPAD, _D2), jnp.float32),
      mesh=mesh,
      scratch_types=[
          pltpu.VMEM_SHARED((_NPAD, _D2), jnp.float32),
          pltpu.VMEM_SHARED((_NPAD, _D2), jnp.float32),
          pltpu.VMEM((_ALLOC2, _C2), jnp.int32),
          pltpu.VMEM((_ALLOC2, _C2), jnp.int32),
          pltpu.VMEM((_C2, _D2), jnp.float32),
          pltpu.VMEM((_C2, _D2), jnp.float32),
          pltpu.SemaphoreType.DMA,
          pltpu.SemaphoreType.DMA,
      ],
      compiler_params=params)
  return agg1, agg2


def _agg2_body(h_hbm, src_hbm, dst_hbm, zeros_hbm, out_hbm,
               hsh, acc, srcv, dstv, buf0, buf1, sem0, sem1):
  c = lax.axis_index("c")
  s = lax.axis_index("s")
  r0 = s * _ROWS_PER_TILE
  pltpu.sync_copy(h_hbm.at[pl.ds(r0, _ROWS_PER_TILE)],
                  hsh.at[pl.ds(r0, _ROWS_PER_TILE)])
  pltpu.sync_copy(zeros_hbm.at[pl.ds(r0, _ROWS_PER_TILE)],
                  acc.at[pl.ds(r0, _ROWS_PER_TILE)])
  pltpu.sync_copy(src_hbm.at[c, s], srcv)
  pltpu.sync_copy(dst_hbm.at[c, s], dstv)
  plsc.subcore_barrier()
  _pipeline(hsh, srcv, dstv, acc, buf0, buf1, sem0, sem1, _NCH2 // 2)
  plsc.subcore_barrier()
  pltpu.sync_copy(acc.at[pl.ds(r0, _ROWS_PER_TILE)],
                  out_hbm.at[c, pl.ds(r0, _ROWS_PER_TILE)])


def _mm_body(a_ref, w1_ref, w2_ref, o_ref):
  wc = jnp.dot(w1_ref[...], w2_ref[...],
               preferred_element_type=jnp.float32,
               precision=lax.Precision.HIGHEST)
  o_ref[...] = (
      jnp.dot(a_ref[0], wc[:_DH],
              preferred_element_type=jnp.float32,
              precision=lax.Precision.HIGHEST)
      + jnp.dot(a_ref[1], wc[_DH:],
                preferred_element_type=jnp.float32,
                precision=lax.Precision.HIGHEST))


_mm = pl.pallas_call(
    _mm_body, out_shape=jax.ShapeDtypeStruct((_NPAD, _D2), jnp.float32))


def _add_body(a_ref, b_ref, o_ref):
  o_ref[...] = a_ref[...] + b_ref[...]


_add = pl.pallas_call(
    _add_body, out_shape=jax.ShapeDtypeStruct((_NPAD, _D2), jnp.float32))


def _edge_layout(src, dst, lead_shape, nch_proc, alloc, chunk):
  n_tiles = 1
  for d in lead_shape:
    n_tiles *= d
  cap = n_tiles * nch_proc * chunk
  npad = cap - src.shape[0]
  src_p = jnp.concatenate(
      [src, jnp.full((npad,), _N, jnp.int32)]).reshape(
          *lead_shape, nch_proc, chunk)
  dst_p = jnp.concatenate(
      [dst, jnp.arange(npad, dtype=jnp.int32) % _NPAD]).reshape(
          *lead_shape, nch_proc, chunk)
  extra = alloc - nch_proc
  src_p = jnp.concatenate(
      [src_p, jnp.full((*lead_shape, extra, chunk), _N, jnp.int32)],
      axis=-2)
  dst_p = jnp.concatenate(
      [dst_p, jnp.zeros((*lead_shape, extra, chunk), jnp.int32)], axis=-2)
  return src_p, dst_p


@jax.jit
def kernel(x, edge_index, W1, W2):
  src = edge_index[0].astype(jnp.int32)
  dst = edge_index[1].astype(jnp.int32)
  x_pad = jnp.zeros((_NPAD, _D1), jnp.float32).at[:_N].set(x)
  x_split = jnp.stack([x_pad[:, :_DH], x_pad[:, _DH:]])

  _agg1, _agg2 = _build_aggs()
  src1, dst1 = _edge_layout(src, dst, (_NSUB,), _NCH1, _ALLOC1, _C1)
  zeros1 = jnp.zeros((_NPAD, _DH), jnp.float32)
  agg1 = _agg1(x_split, src1, dst1, zeros1)

  h2 = _mm(agg1, W1, W2)

  src2, dst2 = _edge_layout(src, dst, (_NCORES, _NSUB), _NCH2, _ALLOC2,
                            _C2)
  zeros2 = jnp.zeros((_NPAD, _D2), jnp.float32)
  parts = _agg2(h2, src2, dst2, zeros2)

  out = _add(parts[0], parts[1])
  return out[:_N]
